# baseline (speedup 1.0000x reference)
import functools

import jax
import jax.numpy as jnp
from jax import lax
from jax.experimental import pallas as pl
from jax.experimental.pallas import tpu as pltpu
from jax.experimental.pallas import tpu_sc as plsc

_DIM = 64
_MAX_LENGTH = 160
_TBL = 2 * _MAX_LENGTH
_NW = 32


def _sc_mesh():
    return plsc.VectorSubcoreMesh(core_axis_name="c", subcore_axis_name="s")


def _build_strip_sc(tbl_flat, seq):
    crows = 2 * seq
    chunk = crows // _NW

    @functools.partial(
        pl.kernel,
        mesh=_sc_mesh(),
        out_type=jax.ShapeDtypeStruct((crows * _DIM,), jnp.float32),
        scratch_types=[
            pltpu.VMEM((_TBL * _DIM,), jnp.float32),
            pltpu.VMEM((chunk * _DIM,), jnp.float32),
        ],
    )
    def build(tbl_hbm, c_hbm, tbl_v, chunk_v):
        wid = lax.axis_index("s") * 2 + lax.axis_index("c")
        pltpu.sync_copy(tbl_hbm, tbl_v)
        base = wid * chunk

        def body(r, carry):
            src = jnp.clip(seq + _MAX_LENGTH - 1 - (base + r), 0, _TBL - 1)
            soff = pl.multiple_of(src * _DIM, _DIM)
            doff = pl.multiple_of(r * _DIM, _DIM)
            for c in range(_DIM // 16):
                chunk_v[pl.ds(doff + c * 16, 16)] = tbl_v[
                    pl.ds(soff + c * 16, 16)]
            return carry

        lax.fori_loop(0, chunk, body, 0)
        pltpu.sync_copy(
            chunk_v, c_hbm.at[pl.ds(pl.multiple_of(base * _DIM, _DIM),
                                    chunk * _DIM)])

    return build(tbl_flat)


def _stream_sc(c_strip, seq):
    rows_pw = seq // _NW
    jt = 1024
    span = jt + rows_pw
    nq = seq // jt

    @functools.partial(
        pl.kernel,
        mesh=_sc_mesh(),
        out_type=jax.ShapeDtypeStruct((seq * seq * _DIM,), jnp.float32),
        scratch_types=[
            pltpu.VMEM((span * _DIM,), jnp.float32),
            pltpu.SemaphoreType.DMA,
        ],
    )
    def stream(c_hbm, out_hbm, buf_v, sem):
        wid = lax.axis_index("s") * 2 + lax.axis_index("c")
        base = wid * rows_pw
        for q in range(nq):
            start = (seq - rows_pw) - base + q * jt
            pltpu.sync_copy(
                c_hbm.at[pl.ds(pl.multiple_of(start * _DIM, _DIM),
                               span * _DIM)],
                buf_v)

            def fire(r, carry):
                soff = pl.multiple_of((rows_pw - 1 - r) * _DIM, _DIM)
                doff = pl.multiple_of(
                    (base + r) * (seq * _DIM) + q * (jt * _DIM), _DIM)
                pltpu.make_async_copy(
                    buf_v.at[pl.ds(soff, jt * _DIM)],
                    out_hbm.at[pl.ds(doff, jt * _DIM)],
                    sem,
                ).start()
                return carry

            lax.fori_loop(0, rows_pw, fire, 0)

            def drain(r, carry):
                soff = pl.multiple_of((rows_pw - 1 - r) * _DIM, _DIM)
                doff = pl.multiple_of(
                    (base + r) * (seq * _DIM) + q * (jt * _DIM), _DIM)
                pltpu.make_async_copy(
                    buf_v.at[pl.ds(soff, jt * _DIM)],
                    out_hbm.at[pl.ds(doff, jt * _DIM)],
                    sem,
                ).wait()
                return carry

            lax.fori_loop(0, rows_pw, drain, 0)

    return stream(c_strip)


def kernel(hidden_states, pe_k_weight):
    seq = hidden_states.shape[1]
    tbl_flat = pe_k_weight.reshape(_TBL * _DIM)
    c_strip = _build_strip_sc(tbl_flat, seq)
    out_flat = _stream_sc(c_strip, seq)
    return out_flat.reshape(seq, seq, _DIM)


# speedup vs baseline: 1.0005x; 1.0005x over previous
import functools

import jax
import jax.numpy as jnp
from jax import lax
from jax.experimental import pallas as pl
from jax.experimental.pallas import tpu as pltpu
from jax.experimental.pallas import tpu_sc as plsc

_DIM = 64
_MAX_LENGTH = 160
_TBL = 2 * _MAX_LENGTH
_NW = 32


def _sc_mesh():
    return plsc.VectorSubcoreMesh(core_axis_name="c", subcore_axis_name="s")


_SC_PARAMS = pltpu.CompilerParams(use_tc_tiling_on_sc=False)


def _build_strip_sc(tbl_flat, seq):
    crows = 2 * seq
    chunk = crows // _NW

    @functools.partial(
        pl.kernel,
        mesh=_sc_mesh(),
        out_type=jax.ShapeDtypeStruct((crows, _DIM), jnp.float32),
        scratch_types=[
            pltpu.VMEM((_TBL * _DIM,), jnp.float32),
            pltpu.VMEM((chunk, _DIM), jnp.float32),
        ],
        compiler_params=_SC_PARAMS,
    )
    def build(tbl_hbm, c_hbm, tbl_v, chunk_v):
        wid = lax.axis_index("s") * 2 + lax.axis_index("c")
        pltpu.sync_copy(tbl_hbm, tbl_v)
        base = wid * chunk

        def body(r, carry):
            src = jnp.clip(seq + _MAX_LENGTH - 1 - (base + r), 0, _TBL - 1)
            soff = pl.multiple_of(src * _DIM, _DIM)
            for c in range(_DIM // 16):
                chunk_v[r, pl.ds(c * 16, 16)] = tbl_v[
                    pl.ds(soff + c * 16, 16)]
            return carry

        lax.fori_loop(0, chunk, body, 0)
        pltpu.sync_copy(chunk_v, c_hbm.at[pl.ds(base, chunk)])

    return build(tbl_flat)


def _stream_sc(c_strip, seq):
    rows_pw = seq // _NW
    jt = 1024
    span = jt + rows_pw
    nq = seq // jt

    @functools.partial(
        pl.kernel,
        mesh=_sc_mesh(),
        out_type=jax.ShapeDtypeStruct((seq, seq, _DIM), jnp.float32),
        scratch_types=[
            pltpu.VMEM((span, _DIM), jnp.float32),
            pltpu.SemaphoreType.DMA,
        ],
        compiler_params=_SC_PARAMS,
    )
    def stream(c_hbm, out_hbm, buf_v, sem):
        wid = lax.axis_index("s") * 2 + lax.axis_index("c")
        base = wid * rows_pw
        for q in range(nq):
            start = (seq - rows_pw) - base + q * jt
            pltpu.sync_copy(c_hbm.at[pl.ds(start, span)], buf_v)

            def fire(r, carry):
                pltpu.make_async_copy(
                    buf_v.at[pl.ds(rows_pw - 1 - r, jt)],
                    out_hbm.at[base + r, pl.ds(q * jt, jt)],
                    sem,
                ).start()
                return carry

            lax.fori_loop(0, rows_pw, fire, 0)

            def drain(r, carry):
                pltpu.make_async_copy(
                    buf_v.at[pl.ds(rows_pw - 1 - r, jt)],
                    out_hbm.at[base + r, pl.ds(q * jt, jt)],
                    sem,
                ).wait()
                return carry

            lax.fori_loop(0, rows_pw, drain, 0)

    return stream(c_strip)


def kernel(hidden_states, pe_k_weight):
    seq = hidden_states.shape[1]
    tbl_flat = pe_k_weight.reshape(_TBL * _DIM)
    c_strip = _build_strip_sc(tbl_flat, seq)
    return _stream_sc(c_strip, seq)


# speedup vs baseline: 2.4065x; 2.4053x over previous
import functools

import jax
import jax.numpy as jnp
from jax.experimental import pallas as pl
from jax.experimental.pallas import tpu as pltpu

_DIM = 64
_MAX_LENGTH = 160
_TBL = 2 * _MAX_LENGTH


def _body(tbl_ref, out_ref, ctv, *, seq):
    pid = pl.program_id(0)

    @pl.when(pid == 0)
    def _build():
        crows = 2 * seq
        s = jax.lax.broadcasted_iota(jnp.int32, (_TBL, crows), 1)
        k = jax.lax.broadcasted_iota(jnp.int32, (_TBL, crows), 0)
        idx = jnp.clip(seq + _MAX_LENGTH - 1 - s, 0, _TBL - 1)
        g = (k == idx).astype(jnp.float32)
        ctv[...] = jax.lax.dot_general(
            tbl_ref[...], g, (((0,), (0,)), ((), ())),
            preferred_element_type=jnp.float32)

    m = seq - 1 - pid
    a = m // 128
    p = m - a * 128
    w = ctv[:, pl.ds(pl.multiple_of(a * 128, 128), seq + 128)]
    rolled = pltpu.roll(w, -p, 1)
    for jt in range(seq // 128):
        sl = rolled[:, 128 * jt:128 * (jt + 1)]
        out_ref[0, :, jt] = sl.reshape(8, 8, 128)


def kernel(hidden_states, pe_k_weight):
    seq = hidden_states.shape[1]
    out = pl.pallas_call(
        functools.partial(_body, seq=seq),
        grid=(seq,),
        in_specs=[pl.BlockSpec((_TBL, _DIM), lambda i: (0, 0))],
        out_specs=pl.BlockSpec(
            (1, 8, seq // 128, 8, 128), lambda i: (i, 0, 0, 0, 0)),
        out_shape=jax.ShapeDtypeStruct(
            (seq, 8, seq // 128, 8, 128), jnp.float32),
        scratch_shapes=[
            pltpu.VMEM((_DIM, 2 * seq), jnp.float32),
        ],
        compiler_params=pltpu.CompilerParams(
            dimension_semantics=("arbitrary",)),
    )(pe_k_weight)
    return out.transpose(0, 2, 4, 1, 3).reshape(seq, seq, _DIM)
